# Initial kernel scaffold; baseline (speedup 1.0000x reference)
import jax
import jax.numpy as jnp
from jax.experimental import pallas as pl


def kernel(entity_emb, drug_emb, relation_emb, edge_index, edge_type, disen_weight_att):
    raise NotImplementedError("write your pallas kernel here")


# speedup vs baseline: 2.3470x; 2.3470x over previous
import functools

import jax
import jax.numpy as jnp
from jax import lax
from jax.experimental import pallas as pl
from jax.experimental.pallas import tpu as pltpu
from jax.experimental.pallas import tpu_sc as plsc

N_ENT = 10000
N_DRUG = 2048
N_RELS = 16
D = 256
DH = 128
N_EDGE = 160000
C = 128
N_CHUNK = N_EDGE // C
NS = 16
CHUNKS_PER_TILE = (N_CHUNK + NS - 1) // NS
ROWS_PAD = 10240
RPT = ROWS_PAD // NS


def _sc_agg(ent_hbm, rel_hbm, head_hbm, tail_hbm, type_hbm, z2_hbm, z1_hbm,
            sums_hbm, cnt_hbm,
            tail_v, type_v, head_v, ent_rows, rel_rows, ones_v, acc_sh, cnt_sh, sem):
    c = lax.axis_index("c")
    s = lax.axis_index("s")
    t0 = s * RPT

    if True:
        pltpu.sync_copy(z2_hbm.at[pl.ds(t0, RPT)], acc_sh.at[pl.ds(t0, RPT)])
        pltpu.sync_copy(z1_hbm.at[pl.ds(t0, RPT)], cnt_sh.at[pl.ds(t0, RPT)])

        def _init_ones(k, carry):
            ones_v[pl.ds(k * 16, 16)] = jnp.ones((16,), jnp.float32)
            return carry
        lax.fori_loop(0, C // 16, _init_ones, 0)
        plsc.subcore_barrier()

        ent_base = c * N_ENT
        rel_base = c * N_RELS

        def chunk_body(i, carry):
            cid = s + i * NS

            @pl.when(cid < N_CHUNK)
            def _():
                base = cid * C
                pltpu.sync_copy(tail_hbm.at[pl.ds(base, C)], tail_v)
                pltpu.sync_copy(type_hbm.at[pl.ds(base, C)], type_v)
                pltpu.sync_copy(head_hbm.at[pl.ds(base, C)], head_v)

                def bias(k, cy):
                    sl = pl.ds(k * 16, 16)
                    tail_v[sl] = tail_v[sl] + ent_base
                    type_v[sl] = type_v[sl] + rel_base
                    return cy
                lax.fori_loop(0, C // 16, bias, 0)

                pltpu.async_copy(ent_hbm.at[tail_v], ent_rows, sem).wait()
                pltpu.async_copy(rel_hbm.at[type_v], rel_rows, sem).wait()

                def mul(e, cy):
                    for j in range(DH // 16):
                        sl = pl.ds(j * 16, 16)
                        ent_rows[e, sl] = ent_rows[e, sl] * rel_rows[e, sl]
                    return cy
                lax.fori_loop(0, C, mul, 0)

                pltpu.sync_copy(ent_rows, acc_sh.at[head_v], add=True)
                pltpu.sync_copy(ones_v, cnt_sh.at[head_v], add=True)
            return carry

        lax.fori_loop(0, CHUNKS_PER_TILE, chunk_body, 0)
        plsc.subcore_barrier()

        pltpu.sync_copy(acc_sh.at[pl.ds(t0, RPT)],
                        sums_hbm.at[pl.ds(c * ROWS_PAD + t0, RPT)])

        @pl.when(c == 0)
        def _():
            pltpu.sync_copy(cnt_sh.at[pl.ds(t0, RPT)], cnt_hbm.at[pl.ds(t0, RPT)])


_agg_call = functools.partial(
    pl.kernel,
    out_type=(jax.ShapeDtypeStruct((2 * ROWS_PAD, DH), jnp.float32),
              jax.ShapeDtypeStruct((ROWS_PAD,), jnp.float32)),
    mesh=plsc.VectorSubcoreMesh(core_axis_name="c", subcore_axis_name="s"),
    scratch_types=[
        pltpu.VMEM((C,), jnp.int32),
        pltpu.VMEM((C,), jnp.int32),
        pltpu.VMEM((C,), jnp.int32),
        pltpu.VMEM((C, DH), jnp.float32),
        pltpu.VMEM((C, DH), jnp.float32),
        pltpu.VMEM((C,), jnp.float32),
        pltpu.VMEM_SHARED((ROWS_PAD, DH), jnp.float32),
        pltpu.VMEM_SHARED((ROWS_PAD,), jnp.float32),
        pltpu.SemaphoreType.DMA,
    ],
)(_sc_agg)


BR = 80


def _tc_div(s0_ref, s1_ref, cnt_ref, out_ref):
    inv = 1.0 / jnp.maximum(cnt_ref[...], 1.0)
    out_ref[:, :DH] = s0_ref[...] * inv
    out_ref[:, DH:] = s1_ref[...] * inv


_div_call = pl.pallas_call(
    _tc_div,
    grid=(N_ENT // BR,),
    in_specs=[
        pl.BlockSpec((BR, DH), lambda i: (i, 0)),
        pl.BlockSpec((BR, DH), lambda i: (ROWS_PAD // BR + i, 0)),
        pl.BlockSpec((BR, 1), lambda i: (i, 0)),
    ],
    out_specs=pl.BlockSpec((BR, D), lambda i: (i, 0)),
    out_shape=jax.ShapeDtypeStruct((N_ENT, D), jnp.float32),
)


def kernel(entity_emb, drug_emb, relation_emb, edge_index, edge_type, disen_weight_att):
    ent_cat = jnp.concatenate([entity_emb[:, :DH], entity_emb[:, DH:]], axis=0)
    rel_cat = jnp.concatenate([relation_emb[:, :DH], relation_emb[:, DH:]], axis=0)
    head = edge_index[0]
    tail = edge_index[1]
    z2 = jnp.zeros((ROWS_PAD, DH), jnp.float32)
    z1 = jnp.zeros((ROWS_PAD,), jnp.float32)

    sums, cnt = _agg_call(ent_cat, rel_cat, head, tail, edge_type, z2, z1)
    entity_agg = _div_call(sums, sums, cnt.reshape(ROWS_PAD, 1))
    return entity_agg, entity_agg[:N_DRUG], relation_emb
